# Initial kernel scaffold; baseline (speedup 1.0000x reference)
import jax
import jax.numpy as jnp
from jax.experimental import pallas as pl


def kernel(x, edge_index, Wl0, bl0, Wr0, g0, b0, Wl1, bl1, Wr1, g1, b1, Wl2, bl2, Wr2):
    raise NotImplementedError("write your pallas kernel here")


# speedup vs baseline: 5.1775x; 5.1775x over previous
import functools

import jax
import jax.numpy as jnp
from jax import lax
from jax.experimental import pallas as pl
from jax.experimental.pallas import tpu as pltpu
from jax.experimental.pallas import tpu_sc as plsc

N = 10000
E = 320000

_NC = 2
_NS = 16
_NW = _NC * _NS
_EPT = E // _NW
_C = 80
_NCHUNK = _EPT // _C
_L = 16
_NPT = 632
_NPAD = 10240
_NB = _NPAD // _NS


@functools.lru_cache(maxsize=None)
def _make_segsum(with_deg):
    D = 128
    mesh = plsc.VectorSubcoreMesh(core_axis_name="c", subcore_axis_name="s")
    out_type = [jax.ShapeDtypeStruct((_NC * N, D), jnp.float32)]
    scratch = [
        pltpu.VMEM((_C,), jnp.int32),
        pltpu.VMEM((_C,), jnp.int32),
        pltpu.VMEM((_C, D), jnp.float32),
        pltpu.VMEM_SHARED((N, D), jnp.float32),
        pltpu.SemaphoreType.DMA,
    ]
    if with_deg:
        out_type.append(jax.ShapeDtypeStruct((_NC * _NPAD,), jnp.float32))
        scratch += [
            pltpu.VMEM((_NPAD,), jnp.float32),
            pltpu.VMEM((_NS, _NB), jnp.float32),
            pltpu.VMEM((_NB,), jnp.float32),
            pltpu.VMEM_SHARED((_NS, _NPAD), jnp.float32),
        ]

    def body(y_hbm, src_hbm, dst_hbm, zD_hbm, *rest):
        if with_deg:
            (acc_out, deg_out, sidx, didx, rows, accsh, sem,
             degloc, dblk, dred, degsh) = rest
        else:
            acc_out, sidx, didx, rows, accsh, sem = rest
        c = lax.axis_index("c")
        s = lax.axis_index("s")
        wid = c * _NS + s
        r0 = pl.multiple_of(jnp.minimum(s * _NPT, N - _NPT), 8)
        pltpu.sync_copy(zD_hbm.at[pl.ds(r0, _NPT)], accsh.at[pl.ds(r0, _NPT)])
        if with_deg:
            zeros = jnp.zeros((_L,), jnp.float32)

            @pl.loop(0, _NPAD // _L)
            def _zero(j):
                degloc[pl.ds(j * _L, _L)] = zeros

        plsc.subcore_barrier()

        e0 = wid * _EPT
        if with_deg:
            ones16 = jnp.ones((_L,), jnp.float32)

        @pl.loop(0, _NCHUNK)
        def _chunk(i):
            base = e0 + i * _C
            pltpu.sync_copy(src_hbm.at[pl.ds(base, _C)], sidx)
            pltpu.sync_copy(dst_hbm.at[pl.ds(base, _C)], didx)
            pltpu.async_copy(y_hbm.at[sidx], rows, sem).wait()
            pltpu.sync_copy(rows, accsh.at[didx], add=True)
            if with_deg:
                for j in range(_C // _L):
                    idxv = didx[pl.ds(j * _L, _L)]
                    plsc.addupdate_scatter(degloc, [idxv], ones16)

        if with_deg:
            pltpu.sync_copy(degloc, degsh.at[s])
        plsc.subcore_barrier()

        off = pl.multiple_of(c * N + r0, 8)
        pltpu.sync_copy(accsh.at[pl.ds(r0, _NPT)], acc_out.at[pl.ds(off, _NPT)])
        if with_deg:
            b0 = pl.multiple_of(s * _NB, 128)
            pltpu.sync_copy(degsh.at[:, pl.ds(b0, _NB)], dblk)

            @pl.loop(0, _NB // _L)
            def _red(k):
                tot = dblk[0, pl.ds(k * _L, _L)]
                for j in range(1, _NS):
                    tot = tot + dblk[j, pl.ds(k * _L, _L)]
                dred[pl.ds(k * _L, _L)] = tot

            doff = pl.multiple_of(c * _NPAD + b0, 8)
            pltpu.sync_copy(dred, deg_out.at[pl.ds(doff, _NB)])

    return pl.kernel(
        body, out_type=out_type, mesh=mesh, scratch_types=scratch,
        compiler_params=pltpu.CompilerParams(needs_layout_passes=False),
    )


def _tc_in_body(x_ref, wl_ref, wr_ref, bl_ref, y_ref, r_ref):
    x = x_ref[...]
    y_ref[...] = jnp.dot(x, wl_ref[...], preferred_element_type=jnp.float32)
    r_ref[...] = (
        jnp.dot(x, wr_ref[...], preferred_element_type=jnp.float32) + bl_ref[...]
    )


def _recip_deg(deg_ref):
    deg = deg_ref[0, pl.ds(0, N), :] + deg_ref[1, pl.ds(0, N), :]
    return 1.0 / jnp.maximum(deg, 1.0)


def _tc_mid_body(acc_ref, deg_ref, r_ref, g_ref, b_ref, wl_ref, wr_ref, bl_ref,
                 y_ref, rn_ref):
    recip = _recip_deg(deg_ref)
    agg = (acc_ref[pl.ds(0, N), :] + acc_ref[pl.ds(N, N), :]) * recip
    h = agg + r_ref[...]
    m = jnp.mean(h, axis=0, keepdims=True)
    d = h - m
    v = jnp.mean(d * d, axis=0, keepdims=True)
    h = g_ref[...] * d * jax.lax.rsqrt(v + 1e-5) + b_ref[...]
    h = jnp.maximum(h, 0.0)
    y_ref[...] = jnp.dot(h, wl_ref[...], preferred_element_type=jnp.float32)
    rn_ref[...] = (
        jnp.dot(h, wr_ref[...], preferred_element_type=jnp.float32) + bl_ref[...]
    )


def _tc_out_body(acc_ref, deg_ref, r_ref, o_ref):
    recip = _recip_deg(deg_ref)
    agg = acc_ref[pl.ds(0, N), pl.ds(0, 64)] + acc_ref[pl.ds(N, N), pl.ds(0, 64)]
    o_ref[...] = agg * recip + r_ref[...]


def _f32(*shapes):
    return [jax.ShapeDtypeStruct(s, jnp.float32) for s in shapes]


_tc_in = pl.pallas_call(_tc_in_body, out_shape=_f32((N, 128), (N, 128)))
_tc_mid128 = pl.pallas_call(_tc_mid_body, out_shape=_f32((N, 128), (N, 128)))
_tc_mid64 = pl.pallas_call(_tc_mid_body, out_shape=_f32((N, 128), (N, 64)))
_tc_out = pl.pallas_call(_tc_out_body, out_shape=_f32((N, 64))[0])


def kernel(x, edge_index, Wl0, bl0, Wr0, g0, b0, Wl1, bl1, Wr1, g1, b1,
           Wl2, bl2, Wr2):
    src = edge_index[0]
    dst = edge_index[1]
    z128 = jnp.zeros((N, 128), jnp.float32)
    Wl2p = jnp.pad(Wl2, ((0, 0), (0, 64)))
    row = lambda a: a.reshape(1, -1)

    y0, r0 = _tc_in(x, Wl0, Wr0, row(bl0))
    acc0, deg_flat = _make_segsum(True)(y0, src, dst, z128)
    degp = deg_flat.reshape(_NC, _NPAD, 1)
    y1, r1 = _tc_mid128(acc0, degp, r0, row(g0), row(b0), Wl1, Wr1, row(bl1))
    (acc1,) = _make_segsum(False)(y1, src, dst, z128)
    y2, r2 = _tc_mid64(acc1, degp, r1, row(g1), row(b1), Wl2p, Wr2, row(bl2))
    (acc2,) = _make_segsum(False)(y2, src, dst, z128)
    return _tc_out(acc2, degp, r2)


# speedup vs baseline: 9.1785x; 1.7728x over previous
import functools

import jax
import jax.numpy as jnp
from jax import lax
from jax.experimental import pallas as pl
from jax.experimental.pallas import tpu as pltpu
from jax.experimental.pallas import tpu_sc as plsc

N = 10000
E = 320000

_NC = 2
_NS = 16
_NW = _NC * _NS
_EPT = E // _NW
_C = 80
_NFULL = 124
_CT = _EPT - _NFULL * _C
_L = 16
_NPT = 632
_NPAD = 10240
_NB = _NPAD // _NS


@functools.lru_cache(maxsize=None)
def _make_segsum(with_deg):
    D = 128
    mesh = plsc.VectorSubcoreMesh(core_axis_name="c", subcore_axis_name="s")
    out_type = [jax.ShapeDtypeStruct((_NC * N, D), jnp.float32)]
    scratch = [
        pltpu.VMEM((2, _C), jnp.int32),
        pltpu.VMEM((2, _C), jnp.int32),
        pltpu.VMEM((2, _C, D), jnp.float32),
        pltpu.VMEM((_CT,), jnp.int32),
        pltpu.VMEM((_CT,), jnp.int32),
        pltpu.VMEM_SHARED((N, D), jnp.float32),
        pltpu.SemaphoreType.DMA,
        pltpu.SemaphoreType.DMA,
    ]
    if with_deg:
        out_type.append(jax.ShapeDtypeStruct((_NC * _NPAD,), jnp.float32))
        scratch += [
            pltpu.VMEM((_NPAD,), jnp.float32),
            pltpu.VMEM((_NB,), jnp.float32),
            pltpu.VMEM((_NB,), jnp.float32),
            pltpu.VMEM_SHARED((_NS, _NPAD), jnp.float32),
        ]

    def body(y_hbm, src_hbm, dst_hbm, zD_hbm, *rest):
        if with_deg:
            (acc_out, deg_out, sidx, didx, rows, tsidx, tdidx, accsh,
             isem, gsem, degloc, dtmp, dred, degsh) = rest
        else:
            (acc_out, sidx, didx, rows, tsidx, tdidx, accsh,
             isem, gsem) = rest
        c = lax.axis_index("c")
        s = lax.axis_index("s")
        wid = c * _NS + s
        r0 = pl.multiple_of(jnp.minimum(s * _NPT, N - _NPT), 8)
        pltpu.sync_copy(zD_hbm.at[pl.ds(r0, _NPT)], accsh.at[pl.ds(r0, _NPT)])
        if with_deg:
            zeros = jnp.zeros((_L,), jnp.float32)

            @pl.loop(0, _NPAD // _L)
            def _zero(j):
                degloc[pl.ds(j * _L, _L)] = zeros

        plsc.subcore_barrier()

        e0 = wid * _EPT
        if with_deg:
            ones16 = jnp.ones((_L,), jnp.float32)

        def idx_start(i, b):
            base = pl.multiple_of(jnp.minimum(e0 + i * _C, E - _C), 8)
            pltpu.make_async_copy(src_hbm.at[pl.ds(base, _C)], sidx.at[b], isem).start()
            pltpu.make_async_copy(dst_hbm.at[pl.ds(base, _C)], didx.at[b], isem).start()

        def idx_wait(b):
            pltpu.make_async_copy(src_hbm.at[pl.ds(0, _C)], sidx.at[b], isem).wait()
            pltpu.make_async_copy(dst_hbm.at[pl.ds(0, _C)], didx.at[b], isem).wait()

        def gather_start(b):
            pltpu.make_async_copy(y_hbm.at[sidx.at[b]], rows.at[b], gsem).start()

        def gather_wait(b):
            pltpu.make_async_copy(y_hbm.at[sidx.at[b]], rows.at[b], gsem).wait()

        def finish_chunk(b):
            pltpu.sync_copy(rows.at[b], accsh.at[didx.at[b]], add=True)
            if with_deg:
                for j in range(_C // _L):
                    idxv = didx[b, pl.ds(j * _L, _L)]
                    plsc.addupdate_scatter(degloc, [idxv], ones16)

        idx_start(0, 0)
        idx_wait(0)
        gather_start(0)
        idx_start(1, 1)

        @pl.loop(0, _NFULL // 2)
        def _pair(g):
            for b in (0, 1):
                i = g * 2 + b
                nb = 1 - b
                gather_wait(b)
                idx_wait(nb)
                gather_start(nb)
                finish_chunk(b)
                idx_start(i + 2, b)

        gather_wait(0)
        idx_wait(0)

        tbase = e0 + _NFULL * _C
        pltpu.sync_copy(src_hbm.at[pl.ds(tbase, _CT)], tsidx)
        pltpu.sync_copy(dst_hbm.at[pl.ds(tbase, _CT)], tdidx)
        pltpu.async_copy(y_hbm.at[tsidx], rows.at[0, pl.ds(0, _CT)], gsem).wait()
        pltpu.sync_copy(rows.at[0, pl.ds(0, _CT)], accsh.at[tdidx], add=True)
        if with_deg:
            for j in range(_CT // _L):
                tidxv = tdidx[pl.ds(j * _L, _L)]
                plsc.addupdate_scatter(degloc, [tidxv], ones16)

        if with_deg:
            pltpu.sync_copy(degloc, degsh.at[s])
        plsc.subcore_barrier()

        off = pl.multiple_of(c * N + r0, 8)
        pltpu.sync_copy(accsh.at[pl.ds(r0, _NPT)], acc_out.at[pl.ds(off, _NPT)])
        if with_deg:
            b0 = pl.multiple_of(s * _NB, 128)
            pltpu.sync_copy(degsh.at[0, pl.ds(b0, _NB)], dred)
            for j in range(1, _NS):
                pltpu.sync_copy(degsh.at[j, pl.ds(b0, _NB)], dtmp)

                @pl.loop(0, _NB // _L)
                def _red(k):
                    sl = pl.ds(k * _L, _L)
                    dred[sl] = dred[sl] + dtmp[sl]

            doff = pl.multiple_of(c * _NPAD + b0, 8)
            pltpu.sync_copy(dred, deg_out.at[pl.ds(doff, _NB)])

    return pl.kernel(
        body, out_type=out_type, mesh=mesh, scratch_types=scratch,
        compiler_params=pltpu.CompilerParams(needs_layout_passes=False),
    )


def _tc_in_body(x_ref, wl_ref, wr_ref, bl_ref, y_ref, r_ref):
    x = x_ref[...]
    y_ref[...] = jnp.dot(x, wl_ref[...], preferred_element_type=jnp.float32)
    r_ref[...] = (
        jnp.dot(x, wr_ref[...], preferred_element_type=jnp.float32) + bl_ref[...]
    )


def _recip_deg(deg_ref):
    deg = deg_ref[0, pl.ds(0, N), :] + deg_ref[1, pl.ds(0, N), :]
    return 1.0 / jnp.maximum(deg, 1.0)


def _tc_mid_body(acc_ref, deg_ref, r_ref, g_ref, b_ref, wl_ref, wr_ref, bl_ref,
                 y_ref, rn_ref):
    recip = _recip_deg(deg_ref)
    agg = (acc_ref[pl.ds(0, N), :] + acc_ref[pl.ds(N, N), :]) * recip
    h = agg + r_ref[...]
    m = jnp.mean(h, axis=0, keepdims=True)
    d = h - m
    v = jnp.mean(d * d, axis=0, keepdims=True)
    h = g_ref[...] * d * jax.lax.rsqrt(v + 1e-5) + b_ref[...]
    h = jnp.maximum(h, 0.0)
    y_ref[...] = jnp.dot(h, wl_ref[...], preferred_element_type=jnp.float32)
    rn_ref[...] = (
        jnp.dot(h, wr_ref[...], preferred_element_type=jnp.float32) + bl_ref[...]
    )


def _tc_out_body(acc_ref, deg_ref, r_ref, o_ref):
    recip = _recip_deg(deg_ref)
    agg = acc_ref[pl.ds(0, N), pl.ds(0, 64)] + acc_ref[pl.ds(N, N), pl.ds(0, 64)]
    o_ref[...] = agg * recip + r_ref[...]


def _f32(*shapes):
    return [jax.ShapeDtypeStruct(s, jnp.float32) for s in shapes]


_tc_in = pl.pallas_call(_tc_in_body, out_shape=_f32((N, 128), (N, 128)))
_tc_mid128 = pl.pallas_call(_tc_mid_body, out_shape=_f32((N, 128), (N, 128)))
_tc_mid64 = pl.pallas_call(_tc_mid_body, out_shape=_f32((N, 128), (N, 64)))
_tc_out = pl.pallas_call(_tc_out_body, out_shape=_f32((N, 64))[0])


def kernel(x, edge_index, Wl0, bl0, Wr0, g0, b0, Wl1, bl1, Wr1, g1, b1,
           Wl2, bl2, Wr2):
    src = edge_index[0]
    dst = edge_index[1]
    z128 = jnp.zeros((N, 128), jnp.float32)
    Wl2p = jnp.pad(Wl2, ((0, 0), (0, 64)))
    row = lambda a: a.reshape(1, -1)

    y0, r0 = _tc_in(x, Wl0, Wr0, row(bl0))
    acc0, deg_flat = _make_segsum(True)(y0, src, dst, z128)
    degp = deg_flat.reshape(_NC, _NPAD, 1)
    y1, r1 = _tc_mid128(acc0, degp, r0, row(g0), row(b0), Wl1, Wr1, row(bl1))
    (acc1,) = _make_segsum(False)(y1, src, dst, z128)
    y2, r2 = _tc_mid64(acc1, degp, r1, row(g1), row(b1), Wl2p, Wr2, row(bl2))
    (acc2,) = _make_segsum(False)(y2, src, dst, z128)
    return _tc_out(acc2, degp, r2)
